# baseline (speedup 1.0000x reference)
import functools

import jax
import jax.numpy as jnp
from jax import lax
from jax.experimental import pallas as pl
from jax.experimental.pallas import tpu as pltpu
from jax.experimental.pallas import tpu_sc as plsc

_T = 1000
_D = 128
_B = 16384
_NC = 2
_NS = 16
_NW = _NC * _NS
_BPW = _B // _NW
_CHUNK = 128
_NCHUNK = _BPW // _CHUNK
_STAGE_SPLIT = [(k * 64, 64) for k in range(15)] + [(960, 40)]

_mesh = plsc.VectorSubcoreMesh(core_axis_name="c", subcore_axis_name="s")


@functools.partial(
    pl.kernel,
    mesh=_mesh,
    out_type=jax.ShapeDtypeStruct((_B, _D), jnp.float32),
    scratch_types=[
        pltpu.VMEM((_BPW,), jnp.int32),
        pltpu.VMEM((_BPW, _D), jnp.float32),
        pltpu.VMEM_SHARED((_T, _D), jnp.float32),
        pltpu.SemaphoreType.DMA,
        pltpu.SemaphoreType.DMA,
    ],
)
def _lookup(te_hbm, t_hbm, out_hbm, idx_v, rows_v, table_s, gsem, wsem):
    sid = lax.axis_index("s")
    wid = sid * _NC + lax.axis_index("c")
    base = wid * _BPW

    for k, (r0, nrows) in enumerate(_STAGE_SPLIT):
        @pl.when(sid == k)
        def _(r0=r0, nrows=nrows):
            pltpu.sync_copy(
                te_hbm.at[pl.ds(r0, nrows)],
                table_s.at[pl.ds(r0, nrows)],
            )

    pltpu.sync_copy(t_hbm.at[pl.ds(base, _BPW)], idx_v)
    plsc.subcore_barrier()

    gathers = []
    for j in range(_NCHUNK):
        gathers.append(
            pltpu.async_copy(
                table_s.at[idx_v.at[pl.ds(j * _CHUNK, _CHUNK)]],
                rows_v.at[pl.ds(j * _CHUNK, _CHUNK)],
                gsem,
            )
        )
    writes = []
    for j in range(_NCHUNK):
        gathers[j].wait()
        writes.append(
            pltpu.async_copy(
                rows_v.at[pl.ds(j * _CHUNK, _CHUNK)],
                out_hbm.at[pl.ds(base + j * _CHUNK, _CHUNK)],
                wsem,
            )
        )
    for c in writes:
        c.wait()


def kernel(te, t):
    if t.dtype != jnp.int32:
        t = t.astype(jnp.int32)
    return _lookup(te, t)
